# baseline (speedup 1.0000x reference)
import functools

import jax
import jax.numpy as jnp
from jax import lax
from jax.experimental import pallas as pl
from jax.experimental.pallas import tpu as pltpu
from jax.experimental.pallas import tpu_sc as plsc

N = 8388608

TC_N = 6291456
SC_N = N - TC_N

NC = 2
NS = 16
L = 16
NW = NC * NS
PER_W = SC_N // NW
CHUNK = 16384
NCH = PER_W // CHUNK
UNROLL = 4

TC_COLS = 1024
TC_BR = 512
TC_GRID = TC_N // (TC_BR * TC_COLS)
TC_ROWS_ALL = N // TC_COLS


def _weight(t):
    a = jnp.abs(t)
    return jnp.where(a > 10.0, 0.0,
                     jnp.where(a > 1.0, 1.0,
                               jnp.where(a > 0.5, 2.0, 5.0)))


def _wd2(p, t):
    d = p - t
    return _weight(t) * (d * d)


def _wsum_chunk(p_ref, t_ref, accs):
    def body(j, accs):
        base = j * (L * UNROLL)
        return tuple(
            accs[u] + _wd2(p_ref[pl.ds(base + u * L, L)],
                           t_ref[pl.ds(base + u * L, L)])
            for u in range(UNROLL))

    return lax.fori_loop(0, CHUNK // (L * UNROLL), body, accs)


def _sc_body(pred_hbm, targ_hbm, out_hbm, p0, t0, p1, t1, accv, sem0, sem1):
    c = lax.axis_index("c")
    s = lax.axis_index("s")
    wid = s * NC + c
    base = TC_N + wid * PER_W

    bufs = ((p0, t0, sem0), (p1, t1, sem1))

    def start(k):
        pb, tb, sem = bufs[k % 2]
        off = base + k * CHUNK
        cp = pltpu.async_copy(pred_hbm.at[pl.ds(off, CHUNK)], pb, sem)
        ct = pltpu.async_copy(targ_hbm.at[pl.ds(off, CHUNK)], tb, sem)
        return cp, ct

    accs = tuple(jnp.zeros((L,), jnp.float32) for _ in range(UNROLL))
    inflight = {0: start(0)}
    for k in range(NCH):
        if k + 1 < NCH:
            inflight[k + 1] = start(k + 1)
        cp, ct = inflight.pop(k)
        cp.wait()
        ct.wait()
        pb, tb, _ = bufs[k % 2]
        accs = _wsum_chunk(pb, tb, accs)

    accv[...] = (accs[0] + accs[1]) + (accs[2] + accs[3])
    pltpu.sync_copy(accv, out_hbm.at[pl.ds(wid * L, L)])


_sc_call = functools.partial(
    pl.kernel,
    mesh=plsc.VectorSubcoreMesh(core_axis_name="c", subcore_axis_name="s"),
    out_type=jax.ShapeDtypeStruct((NW * L,), jnp.float32),
    scratch_types=[
        pltpu.VMEM((CHUNK,), jnp.float32),
        pltpu.VMEM((CHUNK,), jnp.float32),
        pltpu.VMEM((CHUNK,), jnp.float32),
        pltpu.VMEM((CHUNK,), jnp.float32),
        pltpu.VMEM((L,), jnp.float32),
        pltpu.SemaphoreType.DMA,
        pltpu.SemaphoreType.DMA,
    ],
)(_sc_body)


def _tc_body(p_ref, t_ref, out_ref):
    i = pl.program_id(0)

    @pl.when(i == 0)
    def _():
        out_ref[...] = jnp.zeros_like(out_ref)

    x = _wd2(p_ref[...], t_ref[...])
    out_ref[...] += jnp.sum(x, axis=0, keepdims=True)


_tc_call = pl.pallas_call(
    _tc_body,
    grid=(TC_GRID,),
    in_specs=[
        pl.BlockSpec((TC_BR, TC_COLS), lambda i: (i, 0)),
        pl.BlockSpec((TC_BR, TC_COLS), lambda i: (i, 0)),
    ],
    out_specs=pl.BlockSpec((1, TC_COLS), lambda i: (0, 0)),
    out_shape=jax.ShapeDtypeStruct((1, TC_COLS), jnp.float32),
)


def kernel(predicted, target):
    p2 = predicted.reshape(TC_ROWS_ALL, TC_COLS)
    t2 = target.reshape(TC_ROWS_ALL, TC_COLS)
    tc_part = _tc_call(p2, t2)
    sc_part = _sc_call(predicted, target)
    return (jnp.sum(tc_part) + jnp.sum(sc_part)) * (1.0 / 16.0)


# speedup vs baseline: 2.5623x; 2.5623x over previous
import functools

import jax
import jax.numpy as jnp
from jax import lax
from jax.experimental import pallas as pl
from jax.experimental.pallas import tpu as pltpu
from jax.experimental.pallas import tpu_sc as plsc

N = 8388608

TC_N = 6291456
SC_N = N - TC_N

NC = 2
NS = 16
L = 16
NW = NC * NS
PER_W = SC_N // NW
CHUNK = 16384
NCH = PER_W // CHUNK
UNROLL = 4

TC_COLS = 128
TC_BR = 4096
TC_GRID = TC_N // (TC_BR * TC_COLS)
TC_ROWS_ALL = N // TC_COLS


def _weight(t):
    a = jnp.abs(t)
    return jnp.where(a > 10.0, 0.0,
                     jnp.where(a > 1.0, 1.0,
                               jnp.where(a > 0.5, 2.0, 5.0)))


def _wd2(p, t):
    d = p - t
    return _weight(t) * (d * d)


def _wsum_chunk(p_ref, t_ref, accs):
    def body(j, accs):
        base = j * (L * UNROLL)
        return tuple(
            accs[u] + _wd2(p_ref[pl.ds(base + u * L, L)],
                           t_ref[pl.ds(base + u * L, L)])
            for u in range(UNROLL))

    return lax.fori_loop(0, CHUNK // (L * UNROLL), body, accs)


def _sc_body(pred_hbm, targ_hbm, out_hbm, p0, t0, p1, t1, accv, sem0, sem1):
    c = lax.axis_index("c")
    s = lax.axis_index("s")
    wid = s * NC + c
    base = TC_N + wid * PER_W

    bufs = ((p0, t0, sem0), (p1, t1, sem1))

    def start(k):
        pb, tb, sem = bufs[k % 2]
        off = base + k * CHUNK
        cp = pltpu.async_copy(pred_hbm.at[pl.ds(off, CHUNK)], pb, sem)
        ct = pltpu.async_copy(targ_hbm.at[pl.ds(off, CHUNK)], tb, sem)
        return cp, ct

    accs = tuple(jnp.zeros((L,), jnp.float32) for _ in range(UNROLL))
    inflight = {0: start(0)}
    for k in range(NCH):
        if k + 1 < NCH:
            inflight[k + 1] = start(k + 1)
        cp, ct = inflight.pop(k)
        cp.wait()
        ct.wait()
        pb, tb, _ = bufs[k % 2]
        accs = _wsum_chunk(pb, tb, accs)

    accv[...] = (accs[0] + accs[1]) + (accs[2] + accs[3])
    pltpu.sync_copy(accv, out_hbm.at[pl.ds(wid * L, L)])


_sc_call = functools.partial(
    pl.kernel,
    mesh=plsc.VectorSubcoreMesh(core_axis_name="c", subcore_axis_name="s"),
    out_type=jax.ShapeDtypeStruct((NW * L,), jnp.float32),
    scratch_types=[
        pltpu.VMEM((CHUNK,), jnp.float32),
        pltpu.VMEM((CHUNK,), jnp.float32),
        pltpu.VMEM((CHUNK,), jnp.float32),
        pltpu.VMEM((CHUNK,), jnp.float32),
        pltpu.VMEM((L,), jnp.float32),
        pltpu.SemaphoreType.DMA,
        pltpu.SemaphoreType.DMA,
    ],
)(_sc_body)


def _tc_body(p_ref, t_ref, out_ref):
    i = pl.program_id(0)

    @pl.when(i == 0)
    def _():
        out_ref[...] = jnp.zeros_like(out_ref)

    x = _wd2(p_ref[...], t_ref[...])
    out_ref[...] += jnp.sum(x, axis=0, keepdims=True)


_tc_call = pl.pallas_call(
    _tc_body,
    grid=(TC_GRID,),
    in_specs=[
        pl.BlockSpec((TC_BR, TC_COLS), lambda i: (i, 0)),
        pl.BlockSpec((TC_BR, TC_COLS), lambda i: (i, 0)),
    ],
    out_specs=pl.BlockSpec((1, TC_COLS), lambda i: (0, 0)),
    out_shape=jax.ShapeDtypeStruct((1, TC_COLS), jnp.float32),
)


def kernel(predicted, target):
    p2 = predicted.reshape(TC_ROWS_ALL, TC_COLS)
    t2 = target.reshape(TC_ROWS_ALL, TC_COLS)
    tc_part = _tc_call(p2, t2)
    sc_part = _sc_call(predicted, target)
    return (jnp.sum(tc_part) + jnp.sum(sc_part)) * (1.0 / 16.0)
